# Initial kernel scaffold; baseline (speedup 1.0000x reference)
import jax
import jax.numpy as jnp
from jax.experimental import pallas as pl


def kernel(x, emb_table, W, b):
    raise NotImplementedError("write your pallas kernel here")


# speedup vs baseline: 2.3957x; 2.3957x over previous
import functools

import jax
import jax.numpy as jnp
from jax import lax
from jax.experimental import pallas as pl
from jax.experimental.pallas import tpu as pltpu
from jax.experimental.pallas import tpu_sc as plsc


def _pooled_sc(x, emb_table):
    B, H = x.shape
    V, D = emb_table.shape
    L = 16
    NW = 32
    BPW = B // NW
    IW = 100
    RPB = H // IW
    CHUNK = 4
    NBUF = 2
    NCHUNK = BPW // CHUNK
    NACC = 4

    x2 = x.reshape(B * RPB, IW)
    mesh = plsc.VectorSubcoreMesh(core_axis_name="c", subcore_axis_name="s")

    @functools.partial(
        pl.kernel,
        out_type=jax.ShapeDtypeStruct((B, D), jnp.float32),
        mesh=mesh,
        compiler_params=pltpu.CompilerParams(use_tc_tiling_on_sc=False),
        scratch_types=[
            pltpu.VMEM((BPW * RPB, IW), jnp.int32),
            pltpu.VMEM((NBUF, CHUNK * H, D), jnp.float32),
            pltpu.VMEM((BPW, D), jnp.float32),
            pltpu.SemaphoreType.DMA,
            pltpu.SemaphoreType.DMA,
        ],
    )
    def k(x_hbm, tab_hbm, out_hbm, idx_v, gbuf, pooled_v, sem0, sem1):
        wid = lax.axis_index("s") * 2 + lax.axis_index("c")
        row0 = wid * BPW
        sems = [sem0, sem1]

        pltpu.sync_copy(x_hbm.at[pl.ds(row0 * RPB, BPW * RPB)], idx_v)

        def copies(chunk, b):
            out = []
            for r in range(CHUNK):
                for p in range(RPB):
                    j = (chunk * CHUNK + r) * RPB + p
                    out.append((tab_hbm.at[idx_v.at[j]],
                                gbuf.at[b, pl.ds(r * H + p * IW, IW)]))
            return out

        def fire(chunk, b):
            for src, dst in copies(chunk, b):
                pltpu.async_copy(src, dst, sems[b])

        def drain(chunk, b):
            for src, dst in copies(chunk, b):
                pltpu.make_async_copy(src, dst, sems[b]).wait()

        def accumulate(chunk, b):
            base = chunk * CHUNK
            for r in range(CHUNK):
                off = r * H

                def jbody(jj, accs, off=off):
                    j = jj * NACC
                    new = []
                    for kk in range(NACC):
                        a0, a1 = accs[2 * kk], accs[2 * kk + 1]
                        new.append(a0 + gbuf[b, off + j + kk, 0:L])
                        new.append(a1 + gbuf[b, off + j + kk, L:2 * L])
                    return tuple(new)

                zero = jnp.zeros((L,), jnp.float32)
                accs = lax.fori_loop(0, H // NACC, jbody,
                                     tuple(zero for _ in range(2 * NACC)))
                scale = jnp.float32(1.0 / H)
                lo = (accs[0] + accs[2]) + (accs[4] + accs[6])
                hi = (accs[1] + accs[3]) + (accs[5] + accs[7])
                pooled_v[base + r, 0:L] = lo * scale
                pooled_v[base + r, L:2 * L] = hi * scale

        for b in range(NBUF):
            fire(b, b)

        @pl.loop(0, NCHUNK // NBUF - 1)
        def _(oi):
            for b in range(NBUF):
                chunk = oi * NBUF + b
                drain(chunk, b)
                accumulate(chunk, b)
                fire(chunk + NBUF, b)

        for b in range(NBUF):
            chunk = NCHUNK - NBUF + b
            drain(chunk, b)
            accumulate(chunk, b)

        pltpu.sync_copy(pooled_v, out_hbm.at[pl.ds(row0, BPW)])

    return k(x2, emb_table)


def _dense_tc(pooled, W, b):
    B, D = pooled.shape
    O = W.shape[1]

    def body(p_ref, w_ref, b_ref, o_ref):
        logits = jnp.dot(p_ref[...], w_ref[...],
                         preferred_element_type=jnp.float32) + b_ref[...]
        m = jnp.max(logits, axis=1, keepdims=True)
        e = jnp.exp(logits - m)
        o_ref[...] = e / jnp.sum(e, axis=1, keepdims=True)

    return pl.pallas_call(
        body,
        out_shape=jax.ShapeDtypeStruct((B, O), jnp.float32),
    )(pooled, W, b.reshape(1, O))


def kernel(x, emb_table, W, b):
    pooled = _pooled_sc(x, emb_table)
    return _dense_tc(pooled, W, b)


# speedup vs baseline: 2.3978x; 1.0009x over previous
import functools

import jax
import jax.numpy as jnp
from jax import lax
from jax.experimental import pallas as pl
from jax.experimental.pallas import tpu as pltpu
from jax.experimental.pallas import tpu_sc as plsc


def _fmt_idx(xp):
    R, C = xp.shape
    TR = R // 8
    NW = 32
    TPW = TR // NW
    mesh = plsc.VectorSubcoreMesh(core_axis_name="c", subcore_axis_name="s")

    @functools.partial(
        pl.kernel,
        out_type=jax.ShapeDtypeStruct((R * C // 128, 128), jnp.int32),
        mesh=mesh,
        compiler_params=pltpu.CompilerParams(use_tc_tiling_on_sc=True),
        scratch_types=[pltpu.SemaphoreType.DMA],
    )
    def k(xp_h, out_h, sem):
        wid = lax.axis_index("s") * 2 + lax.axis_index("c")

        def copies():
            out = []
            for t in range(TPW):
                for ct in range(C // 128):
                    rt = wid * TPW + t
                    out.append((xp_h.at[pl.ds(8 * rt, 8),
                                        pl.ds(128 * ct, 128)],
                                out_h.at[pl.ds((2 * rt + ct) * 8, 8)]))
            return out

        for src, dst in copies():
            pltpu.async_copy(src, dst, sem)
        for src, dst in copies():
            pltpu.make_async_copy(src, dst, sem).wait()

    return k(xp)


def _pooled_sc(x_view, emb_table):
    B, H = 4096, 200
    V, D = emb_table.shape
    L = 16
    NW = 32
    BPW = B // NW
    IWA, IWB = 128, H - 128
    XR = 2 * BPW
    CHUNK = 4
    NBUF = 2
    NCHUNK = BPW // CHUNK
    NACC = 4

    mesh = plsc.VectorSubcoreMesh(core_axis_name="c", subcore_axis_name="s")

    @functools.partial(
        pl.kernel,
        out_type=jax.ShapeDtypeStruct((B, D), jnp.float32),
        mesh=mesh,
        compiler_params=pltpu.CompilerParams(use_tc_tiling_on_sc=False),
        scratch_types=[
            pltpu.VMEM((XR, 128), jnp.int32),
            pltpu.VMEM((NBUF, CHUNK * H, D), jnp.float32),
            pltpu.VMEM((BPW, D), jnp.float32),
            pltpu.SemaphoreType.DMA,
            pltpu.SemaphoreType.DMA,
        ],
    )
    def k(x_hbm, tab_hbm, out_hbm, idx_v, gbuf, pooled_v, sem0, sem1):
        wid = lax.axis_index("s") * 2 + lax.axis_index("c")
        row0 = wid * BPW
        sems = [sem0, sem1]

        pltpu.sync_copy(x_hbm.at[pl.ds(wid * XR, XR)], idx_v)

        def copies(chunk, b):
            out = []
            for r in range(CHUNK):
                i = chunk * CHUNK + r
                row_a = (i >> 3) * 16 + (i & 7)
                out.append((tab_hbm.at[idx_v.at[row_a]],
                            gbuf.at[b, pl.ds(r * H, IWA)]))
                out.append((tab_hbm.at[idx_v.at[row_a + 8, pl.ds(0, IWB)]],
                            gbuf.at[b, pl.ds(r * H + IWA, IWB)]))
            return out

        def fire(chunk, b):
            for src, dst in copies(chunk, b):
                pltpu.async_copy(src, dst, sems[b])

        def drain(chunk, b):
            for src, dst in copies(chunk, b):
                pltpu.make_async_copy(src, dst, sems[b]).wait()

        def accumulate(chunk, b):
            base = chunk * CHUNK
            for r in range(CHUNK):
                off = r * H

                def jbody(jj, accs, off=off):
                    j = jj * NACC
                    new = []
                    for kk in range(NACC):
                        a0, a1 = accs[2 * kk], accs[2 * kk + 1]
                        new.append(a0 + gbuf[b, off + j + kk, 0:L])
                        new.append(a1 + gbuf[b, off + j + kk, L:2 * L])
                    return tuple(new)

                zero = jnp.zeros((L,), jnp.float32)
                accs = lax.fori_loop(0, H // NACC, jbody,
                                     tuple(zero for _ in range(2 * NACC)))
                scale = jnp.float32(1.0 / H)
                lo = (accs[0] + accs[2]) + (accs[4] + accs[6])
                hi = (accs[1] + accs[3]) + (accs[5] + accs[7])
                pooled_v[base + r, 0:L] = lo * scale
                pooled_v[base + r, L:2 * L] = hi * scale

        for b in range(NBUF):
            fire(b, b)

        @pl.loop(0, NCHUNK // NBUF - 1)
        def _(oi):
            for b in range(NBUF):
                chunk = oi * NBUF + b
                drain(chunk, b)
                accumulate(chunk, b)
                fire(chunk + NBUF, b)

        for b in range(NBUF):
            chunk = NCHUNK - NBUF + b
            drain(chunk, b)
            accumulate(chunk, b)

        pltpu.sync_copy(pooled_v, out_hbm.at[pl.ds(row0, BPW)])

    return k(x_view, emb_table)


def _dense_tc(pooled, W, b):
    B, D = pooled.shape
    O = W.shape[1]

    def body(p_ref, w_ref, b_ref, o_ref):
        logits = jnp.dot(p_ref[...], w_ref[...],
                         preferred_element_type=jnp.float32) + b_ref[...]
        m = jnp.max(logits, axis=1, keepdims=True)
        e = jnp.exp(logits - m)
        o_ref[...] = e / jnp.sum(e, axis=1, keepdims=True)

    return pl.pallas_call(
        body,
        out_shape=jax.ShapeDtypeStruct((B, O), jnp.float32),
    )(pooled, W, b.reshape(1, O))


def kernel(x, emb_table, W, b):
    xp = jnp.pad(x, ((0, 0), (0, 256 - x.shape[1])))
    x_view = _fmt_idx(xp)
    pooled = _pooled_sc(x_view, emb_table)
    return _dense_tc(pooled, W, b)


# speedup vs baseline: 2.4013x; 1.0015x over previous
import functools

import jax
import jax.numpy as jnp
from jax import lax
from jax.experimental import pallas as pl
from jax.experimental.pallas import tpu as pltpu
from jax.experimental.pallas import tpu_sc as plsc


def _sc_embed_pool(x, emb_table):
    B, H = x.shape
    V, D = emb_table.shape
    L = 16
    NW = 32
    BPW = B // NW
    IWA = 128
    IWB = H - IWA
    CHUNK = 4
    NBUF = 2
    NCHUNK = BPW // CHUNK
    NACC = 4

    mesh = plsc.VectorSubcoreMesh(core_axis_name="c", subcore_axis_name="s")

    @functools.partial(
        pl.kernel,
        out_type=jax.ShapeDtypeStruct((B, D), jnp.float32),
        mesh=mesh,
        compiler_params=pltpu.CompilerParams(use_tc_tiling_on_sc=False),
        scratch_types=[
            pltpu.VMEM((BPW, IWA), jnp.int32),
            pltpu.VMEM((BPW, IWB), jnp.int32),
            pltpu.VMEM((NBUF, CHUNK * H, D), jnp.float32),
            pltpu.VMEM((BPW, D), jnp.float32),
            pltpu.SemaphoreType.DMA,
            pltpu.SemaphoreType.DMA,
            pltpu.SemaphoreType.DMA,
        ],
    )
    def k(x_hbm, tab_hbm, out_hbm, idx_a, idx_b, gbuf, pooled_v,
          semi, sem0, sem1):
        wid = lax.axis_index("s") * 2 + lax.axis_index("c")
        row0 = wid * BPW
        sems = [sem0, sem1]

        ica = (x_hbm.at[pl.ds(row0, BPW), pl.ds(0, IWA)], idx_a)
        icb = (x_hbm.at[pl.ds(row0, BPW), pl.ds(IWA, IWB)], idx_b)
        for src, dst in (ica, icb):
            pltpu.async_copy(src, dst, semi)
        for src, dst in (ica, icb):
            pltpu.make_async_copy(src, dst, semi).wait()

        def copies(chunk, b):
            out = []
            for r in range(CHUNK):
                i = chunk * CHUNK + r
                out.append((tab_hbm.at[idx_a.at[i]],
                            gbuf.at[b, pl.ds(r * H, IWA)]))
                out.append((tab_hbm.at[idx_b.at[i]],
                            gbuf.at[b, pl.ds(r * H + IWA, IWB)]))
            return out

        def fire(chunk, b):
            for src, dst in copies(chunk, b):
                pltpu.async_copy(src, dst, sems[b])

        def drain(chunk, b):
            for src, dst in copies(chunk, b):
                pltpu.make_async_copy(src, dst, sems[b]).wait()

        def accumulate(chunk, b):
            base = chunk * CHUNK
            for r in range(CHUNK):
                off = r * H

                def jbody(jj, accs, off=off):
                    j = jj * NACC
                    new = []
                    for kk in range(NACC):
                        a0, a1 = accs[2 * kk], accs[2 * kk + 1]
                        new.append(a0 + gbuf[b, off + j + kk, 0:L])
                        new.append(a1 + gbuf[b, off + j + kk, L:2 * L])
                    return tuple(new)

                zero = jnp.zeros((L,), jnp.float32)
                accs = lax.fori_loop(0, H // NACC, jbody,
                                     tuple(zero for _ in range(2 * NACC)))
                scale = jnp.float32(1.0 / H)
                lo = (accs[0] + accs[2]) + (accs[4] + accs[6])
                hi = (accs[1] + accs[3]) + (accs[5] + accs[7])
                pooled_v[base + r, 0:L] = lo * scale
                pooled_v[base + r, L:2 * L] = hi * scale

        for b in range(NBUF):
            fire(b, b)

        @pl.loop(0, NCHUNK // NBUF - 1)
        def _(oi):
            for b in range(NBUF):
                chunk = oi * NBUF + b
                drain(chunk, b)
                accumulate(chunk, b)
                fire(chunk + NBUF, b)

        for b in range(NBUF):
            chunk = NCHUNK - NBUF + b
            drain(chunk, b)
            accumulate(chunk, b)

        pltpu.sync_copy(pooled_v, out_hbm.at[pl.ds(row0, BPW)])

    return k(x, emb_table)


def _dense_tc(pooled, W, b):
    B, D = pooled.shape
    O = W.shape[1]

    def body(p_ref, w_ref, b_ref, o_ref):
        logits = jnp.dot(p_ref[...], w_ref[...],
                         preferred_element_type=jnp.float32) + b_ref[...]
        m = jnp.max(logits, axis=1, keepdims=True)
        e = jnp.exp(logits - m)
        o_ref[...] = e / jnp.sum(e, axis=1, keepdims=True)

    return pl.pallas_call(
        body,
        out_shape=jax.ShapeDtypeStruct((B, O), jnp.float32),
    )(pooled, W, b.reshape(1, O))


def kernel(x, emb_table, W, b):
    pooled = _sc_embed_pool(x, emb_table)
    return _dense_tc(pooled, W, b)


# speedup vs baseline: 2.4146x; 1.0055x over previous
import functools

import jax
import jax.numpy as jnp
from jax import lax
from jax.experimental import pallas as pl
from jax.experimental.pallas import tpu as pltpu
from jax.experimental.pallas import tpu_sc as plsc


def _sc_embed_pool(xv, emb_table, H):
    B = xv.shape[0] // 2
    V, D = emb_table.shape
    L = 16
    NW = 32
    BPW = B // NW
    IWA = 128
    IWB = H - IWA
    CHUNK = 4
    NBUF = 2
    NCHUNK = BPW // CHUNK
    NACC = 4

    mesh = plsc.VectorSubcoreMesh(core_axis_name="c", subcore_axis_name="s")

    @functools.partial(
        pl.kernel,
        out_type=jax.ShapeDtypeStruct((B, D), jnp.float32),
        mesh=mesh,
        compiler_params=pltpu.CompilerParams(use_tc_tiling_on_sc=False),
        scratch_types=[
            pltpu.VMEM((2 * BPW, 128), jnp.int32),
            pltpu.VMEM((NBUF, CHUNK * H, D), jnp.float32),
            pltpu.VMEM((BPW, D), jnp.float32),
            pltpu.SemaphoreType.DMA,
            pltpu.SemaphoreType.DMA,
            pltpu.SemaphoreType.DMA,
        ],
    )
    def k(x_hbm, tab_hbm, out_hbm, idx_v, gbuf, pooled_v,
          semi, sem0, sem1):
        wid = lax.axis_index("s") * 2 + lax.axis_index("c")
        row0 = wid * BPW
        sems = [sem0, sem1]

        pltpu.async_copy(x_hbm.at[pl.ds(wid * 2 * BPW, 2 * BPW)], idx_v,
                         semi)
        pltpu.make_async_copy(x_hbm.at[pl.ds(wid * 2 * BPW, 2 * BPW)],
                              idx_v, semi).wait()

        def copies(chunk, b):
            out = []
            for r in range(CHUNK):
                i = chunk * CHUNK + r
                out.append((tab_hbm.at[idx_v.at[2 * i]],
                            gbuf.at[b, pl.ds(r * H, IWA)]))
                out.append((tab_hbm.at[idx_v.at[2 * i + 1, pl.ds(0, IWB)]],
                            gbuf.at[b, pl.ds(r * H + IWA, IWB)]))
            return out

        def fire(chunk, b):
            for src, dst in copies(chunk, b):
                pltpu.async_copy(src, dst, sems[b])

        def drain(chunk, b):
            for src, dst in copies(chunk, b):
                pltpu.make_async_copy(src, dst, sems[b]).wait()

        def accumulate(chunk, b):
            base = chunk * CHUNK
            for r in range(CHUNK):
                off = r * H

                def jbody(jj, accs, off=off):
                    j = jj * NACC
                    new = []
                    for kk in range(NACC):
                        a0, a1 = accs[2 * kk], accs[2 * kk + 1]
                        new.append(a0 + gbuf[b, off + j + kk, 0:L])
                        new.append(a1 + gbuf[b, off + j + kk, L:2 * L])
                    return tuple(new)

                zero = jnp.zeros((L,), jnp.float32)
                accs = lax.fori_loop(0, H // NACC, jbody,
                                     tuple(zero for _ in range(2 * NACC)))
                scale = jnp.float32(1.0 / H)
                lo = (accs[0] + accs[2]) + (accs[4] + accs[6])
                hi = (accs[1] + accs[3]) + (accs[5] + accs[7])
                pooled_v[base + r, 0:L] = lo * scale
                pooled_v[base + r, L:2 * L] = hi * scale

        for b in range(NBUF):
            fire(b, b)

        @pl.loop(0, NCHUNK // NBUF - 1)
        def _(oi):
            for b in range(NBUF):
                chunk = oi * NBUF + b
                drain(chunk, b)
                accumulate(chunk, b)
                fire(chunk + NBUF, b)

        for b in range(NBUF):
            chunk = NCHUNK - NBUF + b
            drain(chunk, b)
            accumulate(chunk, b)

        pltpu.sync_copy(pooled_v, out_hbm.at[pl.ds(row0, BPW)])

    return k(xv, emb_table)


def _dense_tc(pooled, W, b):
    B, D = pooled.shape
    O = W.shape[1]

    def body(p_ref, w_ref, b_ref, o_ref):
        logits = jnp.dot(p_ref[...], w_ref[...],
                         preferred_element_type=jnp.float32) + b_ref[...]
        m = jnp.max(logits, axis=1, keepdims=True)
        e = jnp.exp(logits - m)
        o_ref[...] = e / jnp.sum(e, axis=1, keepdims=True)

    return pl.pallas_call(
        body,
        out_shape=jax.ShapeDtypeStruct((B, O), jnp.float32),
    )(pooled, W, b.reshape(1, O))


def kernel(x, emb_table, W, b):
    B, H = x.shape
    xv = jnp.pad(x, ((0, 0), (0, 256 - H))).reshape(2 * B, 128)
    pooled = _sc_embed_pool(xv, emb_table, H)
    return _dense_tc(pooled, W, b)
